# baseline (speedup 1.0000x reference)
import functools

import jax
import jax.numpy as jnp
from jax import lax
from jax.experimental import pallas as pl
from jax.experimental.pallas import tpu as pltpu
from jax.experimental.pallas import tpu_sc as plsc

N_USERS = 50000
N_ITEMS = 50000
N_NODES = N_USERS + N_ITEMS
K = 64
L = 16
EBLK = 128
NSEG = 4
N_PAD = 100096
ROWBLK = 2000


def _sc_mesh():
    return plsc.VectorSubcoreMesh(core_axis_name="c", subcore_axis_name="s")


def _worker_geometry():
    info = plsc.get_sparse_core_info()
    return info.num_cores, info.num_subcores


def _make_deg_kernel(nc, ns, e_pad):
    e_core = e_pad // nc
    e_sub = e_core // ns
    nblk = e_sub // EBLK
    rows_per_sub = N_PAD // ns

    out_type = [jax.ShapeDtypeStruct((N_PAD, L), jnp.float32) for _ in range(nc)]

    @functools.partial(
        pl.kernel,
        mesh=_sc_mesh(),
        out_type=out_type,
        compiler_params=pltpu.CompilerParams(use_tc_tiling_on_sc=False),
        scratch_types=[
            pltpu.VMEM((e_pad // (nc * ns) // EBLK // NSEG, EBLK), jnp.int32),
            pltpu.VMEM((EBLK, L), jnp.float32),
            pltpu.VMEM_SHARED((N_PAD, L), jnp.float32),
            pltpu.SemaphoreType.DMA,
        ],
    )
    def deg_kernel(dst_hbm, ones_hbm, zeros_hbm, *rest):
        nseg_blk = nblk // NSEG
        outs = rest[:nc]
        dst_all, ones_v, shared, sem = rest[nc:]
        cid = lax.axis_index("c")
        sid = lax.axis_index("s")
        pltpu.sync_copy(
            zeros_hbm.at[pl.ds(sid * rows_per_sub, rows_per_sub)],
            shared.at[pl.ds(sid * rows_per_sub, rows_per_sub)],
        )
        base_blk = (cid * e_core + sid * e_sub) // EBLK
        pltpu.sync_copy(ones_hbm, ones_v)
        plsc.subcore_barrier()

        def body(j, carry):
            pltpu.sync_copy(ones_v, shared.at[dst_all.at[j]], add=True)
            return carry

        for sg in range(NSEG):
            pltpu.sync_copy(
                dst_hbm.at[pl.ds(base_blk + sg * nseg_blk, nseg_blk)], dst_all)
            lax.fori_loop(0, nseg_blk, body, 0)
        plsc.subcore_barrier()
        for c in range(nc):
            @pl.when(cid == c)
            def _():
                pltpu.sync_copy(
                    shared.at[pl.ds(sid * rows_per_sub, rows_per_sub)],
                    outs[c].at[pl.ds(sid * rows_per_sub, rows_per_sub)],
                )

    return deg_kernel


def _make_segsum_kernel(nc, ns, e_pad, nchunks):
    e_core = e_pad // nc
    e_sub = e_core // ns
    nblk = e_sub // EBLK
    rows_per_sub = N_PAD // ns

    out_type = [
        jax.ShapeDtypeStruct((N_PAD, L), jnp.float32) for _ in range(nc * nchunks)
    ]

    @functools.partial(
        pl.kernel,
        mesh=_sc_mesh(),
        out_type=out_type,
        compiler_params=pltpu.CompilerParams(use_tc_tiling_on_sc=False),
        scratch_types=[
            pltpu.VMEM((e_pad // (nc * ns) // EBLK // NSEG, EBLK), jnp.int32),
            pltpu.VMEM((e_pad // (nc * ns) // EBLK // NSEG, EBLK), jnp.int32),
            pltpu.VMEM((EBLK, L), jnp.float32),
            pltpu.VMEM((EBLK, L), jnp.float32),
            pltpu.VMEM((EBLK, L), jnp.float32),
            pltpu.VMEM_SHARED((N_PAD, L), jnp.float32),
            pltpu.SemaphoreType.DMA,
            pltpu.SemaphoreType.DMA,
            pltpu.SemaphoreType.DMA,
            pltpu.SemaphoreType.DMA,
            pltpu.SemaphoreType.DMA,
            pltpu.SemaphoreType.DMA,
        ],
    )
    def segsum_kernel(*refs):
        nseg_blk = nblk // NSEG
        tables = refs[:nchunks]
        src_hbm, dst_hbm, zeros_hbm = refs[nchunks : nchunks + 3]
        outs = refs[nchunks + 3 : nchunks + 3 + nc * nchunks]
        scratch = refs[nchunks + 3 + nc * nchunks :]
        src_all, dst_all = scratch[0:2]
        rows = scratch[2:5]
        shared = scratch[5]
        gsems = scratch[6:9]
        ssems = scratch[9:12]
        cid = lax.axis_index("c")
        sid = lax.axis_index("s")
        base_blk = (cid * e_core + sid * e_sub) // EBLK

        for ch in range(nchunks):
            pltpu.sync_copy(
                zeros_hbm.at[pl.ds(sid * rows_per_sub, rows_per_sub)],
                shared.at[pl.ds(sid * rows_per_sub, rows_per_sub)],
            )
            plsc.subcore_barrier()

            def fire_g(j, b):
                pltpu.async_copy(tables[ch].at[src_all.at[j]], rows[b], gsems[b])

            def drain_g(j, b):
                pltpu.make_async_copy(
                    tables[ch].at[src_all.at[j]], rows[b], gsems[b]).wait()

            def fire_s(j, b):
                pltpu.async_copy(rows[b], shared.at[dst_all.at[j]], ssems[b],
                                 add=True)

            def drain_s(j, b):
                pltpu.make_async_copy(
                    rows[b], shared.at[dst_all.at[j]], ssems[b]).wait()

            for sg in range(NSEG):
                pltpu.sync_copy(
                    src_hbm.at[pl.ds(base_blk + sg * nseg_blk, nseg_blk)],
                    src_all)
                pltpu.sync_copy(
                    dst_hbm.at[pl.ds(base_blk + sg * nseg_blk, nseg_blk)],
                    dst_all)
                fire_g(0, 0)
                fire_g(1, 1)
                fire_g(2, 2)
                drain_g(0, 0)
                fire_s(0, 0)

                def body(t, carry):
                    j0 = 1 + 3 * t
                    for k in range(3):
                        j = j0 + k
                        b = (1 + k) % 3
                        drain_g(j, b)
                        fire_s(j, b)
                        drain_s(j - 1, k % 3)
                        fire_g(j + 2, k % 3)
                    return carry

                lax.fori_loop(0, (nseg_blk - 3) // 3, body, 0)
                for j in (nseg_blk - 2, nseg_blk - 1):
                    drain_g(j, j % 3)
                    fire_s(j, j % 3)
                for j in (nseg_blk - 3, nseg_blk - 2, nseg_blk - 1):
                    drain_s(j, j % 3)
            plsc.subcore_barrier()
            for c in range(nc):
                @pl.when(cid == c)
                def _():
                    pltpu.sync_copy(
                        shared.at[pl.ds(sid * rows_per_sub, rows_per_sub)],
                        outs[c * nchunks + ch].at[pl.ds(sid * rows_per_sub, rows_per_sub)],
                    )
            plsc.subcore_barrier()

    return segsum_kernel


def _make_batch_gather_kernel(nc, ns, batch, ntab):
    nw = nc * ns
    b_per_w = batch // nw

    out_type = [
        jax.ShapeDtypeStruct((batch, K), jnp.float32) for _ in range(2 * ntab)
    ]

    @functools.partial(
        pl.kernel,
        mesh=_sc_mesh(),
        out_type=out_type,
        compiler_params=pltpu.CompilerParams(use_tc_tiling_on_sc=False),
        scratch_types=[
            pltpu.VMEM((b_per_w,), jnp.int32),
            pltpu.VMEM((b_per_w, K), jnp.float32),
            pltpu.SemaphoreType.DMA,
        ],
    )
    def gather_kernel(*refs):
        tables = refs[:ntab]
        uidx_hbm, iidx_hbm = refs[ntab : ntab + 2]
        outs = refs[ntab + 2 : ntab + 2 + 2 * ntab]
        idx_v, rows_v, sem = refs[ntab + 2 + 2 * ntab :]
        cid = lax.axis_index("c")
        sid = lax.axis_index("s")
        wid = sid * nc + cid
        base = wid * b_per_w
        pltpu.sync_copy(uidx_hbm.at[pl.ds(base, b_per_w)], idx_v)
        for t in range(ntab):
            pltpu.async_copy(tables[t].at[idx_v], rows_v, sem).wait()
            pltpu.sync_copy(rows_v, outs[t].at[pl.ds(base, b_per_w)])
        pltpu.sync_copy(iidx_hbm.at[pl.ds(base, b_per_w)], idx_v)
        for t in range(ntab):
            pltpu.async_copy(tables[t].at[idx_v], rows_v, sem).wait()
            pltpu.sync_copy(rows_v, outs[ntab + t].at[pl.ds(base, b_per_w)])

    return gather_kernel


def _scale_body(x_ref, d0_ref, d1_ref, dinv_ref, *y_refs):
    deg = jnp.maximum(d0_ref[...] + d1_ref[...], 1.0)
    dinv = lax.rsqrt(deg)
    dinv_ref[...] = dinv
    y = x_ref[...] * dinv[:, 0:1]
    for c in range(K // L):
        y_refs[c][...] = y[:, c * L:(c + 1) * L]


def _tc_scale(x0, d0, d1):
    grid = N_NODES // ROWBLK
    rb = lambda i: (i, 0)
    return pl.pallas_call(
        _scale_body,
        grid=(grid,),
        in_specs=[
            pl.BlockSpec((ROWBLK, K), rb),
            pl.BlockSpec((ROWBLK, L), rb),
            pl.BlockSpec((ROWBLK, L), rb),
        ],
        out_specs=[pl.BlockSpec((ROWBLK, L), rb)] * 5,
        out_shape=[jax.ShapeDtypeStruct((N_NODES, L), jnp.float32)] * 5,
    )(x0, d0, d1)


def _make_layer_body(emit_y):
    def body(x_ref, dinv_ref, p00, p01, p02, p03, p10, p11, p12, p13,
             pn0, pn1, w1_ref, w2_ref, bias_ref, *out_refs):
        x = x_ref[...]
        dinvcol = dinv_ref[...][:, 0:1]
        s1 = jnp.concatenate(
            [p00[...] + p10[...], p01[...] + p11[...],
             p02[...] + p12[...], p03[...] + p13[...]], axis=1)
        s1 = s1 * dinvcol
        nsum = (pn0[...][:, 0:1] + pn1[...][:, 0:1]) * dinvcol
        b1 = bias_ref[...][0:1, :]
        bsum = bias_ref[...][1:2, :]
        t = (
            jnp.dot(s1 + x, w1_ref[...], preferred_element_type=jnp.float32)
            + jnp.dot(x * s1, w2_ref[...], preferred_element_type=jnp.float32)
            + nsum * bsum
            + b1
        )
        xn = jnp.where(t >= 0.0, t, 0.2 * t)
        out_refs[0][...] = xn
        if emit_y:
            yn = xn * dinvcol
            for c in range(K // L):
                out_refs[1 + c][...] = yn[:, c * L:(c + 1) * L]
    return body


def _tc_layer(x, dinv16, parts, pn, W1, W2, bias_pack, emit_y):
    grid = N_NODES // ROWBLK
    rb = lambda i: (i, 0)
    zb = lambda i: (0, 0)
    in_specs = (
        [pl.BlockSpec((ROWBLK, K), rb), pl.BlockSpec((ROWBLK, L), rb)]
        + [pl.BlockSpec((ROWBLK, L), rb)] * 10
        + [pl.BlockSpec((K, K), zb)] * 2
        + [pl.BlockSpec((8, K), zb)]
    )
    out_specs = [pl.BlockSpec((ROWBLK, K), rb)]
    out_shape = [jax.ShapeDtypeStruct((N_NODES, K), jnp.float32)]
    if emit_y:
        out_specs += [pl.BlockSpec((ROWBLK, L), rb)] * (K // L)
        out_shape += [jax.ShapeDtypeStruct((N_NODES, L), jnp.float32)] * (K // L)
    res = pl.pallas_call(
        _make_layer_body(emit_y),
        grid=(grid,),
        in_specs=in_specs,
        out_specs=out_specs,
        out_shape=out_shape,
    )(x, dinv16, *parts, *pn, W1, W2, bias_pack)
    return res


def _dot_body(*refs):
    gus = refs[0:3]
    gis = refs[3:6]
    out_ref = refs[6]
    acc = gus[0][...] * gis[0][...]
    acc = acc + gus[1][...] * gis[1][...]
    acc = acc + gus[2][...] * gis[2][...]
    s = jnp.sum(acc, axis=1, keepdims=True)
    out_ref[...] = jnp.broadcast_to(s, out_ref.shape)


def _tc_dot(gus, gis, batch):
    return pl.pallas_call(
        _dot_body,
        grid=(1,),
        in_specs=[pl.BlockSpec((batch, K), lambda i: (0, 0))] * 6,
        out_specs=pl.BlockSpec((batch, 128), lambda i: (0, 0)),
        out_shape=jax.ShapeDtypeStruct((batch, 128), jnp.float32),
    )(*gus, *gis)


def kernel(Gu, Gi, W1_0, b1_0, W2_0, b2_0, W1_1, b1_1, W2_1, b2_1,
           edge_index, user, item):
    nc, ns = _worker_geometry()
    nw = nc * ns
    n_edges = edge_index.shape[1]
    e_sym = 2 * n_edges
    blk_all = nw * EBLK * 3 * NSEG
    e_pad = ((e_sym + blk_all - 1) // blk_all) * blk_all

    u = edge_index[0]
    i_n = edge_index[1] + N_USERS
    src = jnp.concatenate([u, i_n])
    dst = jnp.concatenate([i_n, u])
    pad = e_pad - e_sym
    src_p = jnp.concatenate([src, jnp.zeros((pad,), jnp.int32)]).reshape(-1, EBLK)
    dst_p = jnp.concatenate(
        [dst, jnp.full((pad,), N_NODES, jnp.int32)]).reshape(-1, EBLK)

    zeros_buf = jnp.zeros((N_PAD, L), jnp.float32)
    ones_blk = jnp.ones((EBLK, L), jnp.float32)

    deg_parts = _make_deg_kernel(nc, ns, e_pad)(dst_p, ones_blk, zeros_buf)
    d0 = deg_parts[0]
    d1 = deg_parts[1] if nc > 1 else jnp.zeros_like(d0)
    for c in range(2, nc):
        d1 = d1 + deg_parts[c]

    x0 = jnp.concatenate([Gu, Gi], axis=0)
    dinv16, *y0_chunks = _tc_scale(x0, d0, d1)

    bias_pack0 = jnp.zeros((8, K), jnp.float32).at[0].set(b1_0).at[1].set(b1_0 + b2_0)
    bias_pack1 = jnp.zeros((8, K), jnp.float32).at[0].set(b1_1).at[1].set(b1_1 + b2_1)

    seg5 = _make_segsum_kernel(nc, ns, e_pad, 5)
    parts = seg5(*y0_chunks, dinv16, src_p, dst_p, zeros_buf)
    def pick(parts, nchunks, ch):
        p0 = parts[ch]
        p1 = (parts[nchunks + ch] if nc > 1 else jnp.zeros_like(p0))
        for c in range(2, nc):
            p1 = p1 + parts[c * nchunks + ch]
        return p0, p1

    p_feat0 = [pick(parts, 5, ch)[0] for ch in range(4)]
    p_feat1 = [pick(parts, 5, ch)[1] for ch in range(4)]
    pn0, pn1 = pick(parts, 5, 4)

    x1, *y1_chunks = _tc_layer(
        x0, dinv16, p_feat0 + p_feat1, [pn0, pn1], W1_0, W2_0, bias_pack0, True)

    seg4 = _make_segsum_kernel(nc, ns, e_pad, 4)
    parts2 = seg4(*y1_chunks, src_p, dst_p, zeros_buf)
    q_feat0 = [pick(parts2, 4, ch)[0] for ch in range(4)]
    q_feat1 = [pick(parts2, 4, ch)[1] for ch in range(4)]

    (x2,) = _tc_layer(
        x1, dinv16, q_feat0 + q_feat1, [pn0, pn1], W1_1, W2_1, bias_pack1, False)

    batch = user.shape[0]
    gath = _make_batch_gather_kernel(nc, ns, batch, 3)
    g = gath(x0, x1, x2, user, item + N_USERS)
    gus, gis = g[0:3], g[3:6]
    out = _tc_dot(gus, gis, batch)
    return out[:, 0]


# speedup vs baseline: 1.1529x; 1.1529x over previous
import functools

import jax
import jax.numpy as jnp
from jax import lax
from jax.experimental import pallas as pl
from jax.experimental.pallas import tpu as pltpu
from jax.experimental.pallas import tpu_sc as plsc

N_USERS = 50000
N_ITEMS = 50000
N_NODES = N_USERS + N_ITEMS
K = 64
L = 16
EBLK = 128
NSEG = 4
N_PAD = 100096
ROWBLK = 2000


def _sc_mesh():
    return plsc.VectorSubcoreMesh(core_axis_name="c", subcore_axis_name="s")


def _worker_geometry():
    info = plsc.get_sparse_core_info()
    return info.num_cores, info.num_subcores


def _make_deg_kernel(nc, ns, e_pad):
    e_core = e_pad // nc
    e_sub = e_core // ns
    nblk = e_sub // EBLK
    rows_per_sub = N_PAD // ns

    out_type = [jax.ShapeDtypeStruct((N_PAD, L), jnp.float32) for _ in range(nc)]

    @functools.partial(
        pl.kernel,
        mesh=_sc_mesh(),
        out_type=out_type,
        compiler_params=pltpu.CompilerParams(use_tc_tiling_on_sc=False),
        scratch_types=[
            pltpu.VMEM((e_pad // (nc * ns) // EBLK // NSEG, EBLK), jnp.int32),
            pltpu.VMEM((EBLK, L), jnp.float32),
            pltpu.VMEM_SHARED((N_PAD, L), jnp.float32),
            pltpu.SemaphoreType.DMA,
        ],
    )
    def deg_kernel(dst_hbm, ones_hbm, zeros_hbm, *rest):
        nseg_blk = nblk // NSEG
        outs = rest[:nc]
        dst_all, ones_v, shared, sem = rest[nc:]
        cid = lax.axis_index("c")
        sid = lax.axis_index("s")
        pltpu.sync_copy(
            zeros_hbm.at[pl.ds(sid * rows_per_sub, rows_per_sub)],
            shared.at[pl.ds(sid * rows_per_sub, rows_per_sub)],
        )
        base_blk = (cid * e_core + sid * e_sub) // EBLK
        pltpu.sync_copy(ones_hbm, ones_v)
        plsc.subcore_barrier()

        def body(j, carry):
            pltpu.sync_copy(ones_v, shared.at[dst_all.at[j]], add=True)
            return carry

        for sg in range(NSEG):
            pltpu.sync_copy(
                dst_hbm.at[pl.ds(base_blk + sg * nseg_blk, nseg_blk)], dst_all)
            lax.fori_loop(0, nseg_blk, body, 0)
        plsc.subcore_barrier()
        for c in range(nc):
            @pl.when(cid == c)
            def _():
                pltpu.sync_copy(
                    shared.at[pl.ds(sid * rows_per_sub, rows_per_sub)],
                    outs[c].at[pl.ds(sid * rows_per_sub, rows_per_sub)],
                )

    return deg_kernel


def _make_segsum_kernel(nc, ns, e_pad, nchunks):
    e_core = e_pad // nc
    e_sub = e_core // ns
    nblk = e_sub // EBLK
    rows_per_sub = N_PAD // ns

    out_type = [
        jax.ShapeDtypeStruct((N_PAD, L), jnp.float32) for _ in range(nc * nchunks)
    ]

    @functools.partial(
        pl.kernel,
        mesh=_sc_mesh(),
        out_type=out_type,
        compiler_params=pltpu.CompilerParams(use_tc_tiling_on_sc=False),
        scratch_types=[
            pltpu.VMEM((e_pad // (nc * ns) // EBLK // NSEG, EBLK), jnp.int32),
            pltpu.VMEM((e_pad // (nc * ns) // EBLK // NSEG, EBLK), jnp.int32),
            pltpu.VMEM((EBLK, L), jnp.float32),
            pltpu.VMEM((EBLK, L), jnp.float32),
            pltpu.VMEM_SHARED((N_PAD, L), jnp.float32),
            pltpu.SemaphoreType.DMA,
            pltpu.SemaphoreType.DMA,
        ],
    )
    def segsum_kernel(*refs):
        nseg_blk = nblk // NSEG
        tables = refs[:nchunks]
        src_hbm, dst_hbm, zeros_hbm = refs[nchunks : nchunks + 3]
        outs = refs[nchunks + 3 : nchunks + 3 + nc * nchunks]
        scratch = refs[nchunks + 3 + nc * nchunks :]
        src_all, dst_all, rows0, rows1, shared, sem0, sem1 = scratch
        rows = (rows0, rows1)
        sems = (sem0, sem1)
        cid = lax.axis_index("c")
        sid = lax.axis_index("s")
        base_blk = (cid * e_core + sid * e_sub) // EBLK

        for ch in range(nchunks):
            pltpu.sync_copy(
                zeros_hbm.at[pl.ds(sid * rows_per_sub, rows_per_sub)],
                shared.at[pl.ds(sid * rows_per_sub, rows_per_sub)],
            )
            plsc.subcore_barrier()

            def fire(j, b):
                pltpu.async_copy(tables[ch].at[src_all.at[j]], rows[b], sems[b])

            def drain(j, b):
                pltpu.make_async_copy(
                    tables[ch].at[src_all.at[j]], rows[b], sems[b]).wait()

            def scat(j, b):
                pltpu.sync_copy(rows[b], shared.at[dst_all.at[j]], add=True)

            for sg in range(NSEG):
                pltpu.sync_copy(
                    src_hbm.at[pl.ds(base_blk + sg * nseg_blk, nseg_blk)],
                    src_all)
                pltpu.sync_copy(
                    dst_hbm.at[pl.ds(base_blk + sg * nseg_blk, nseg_blk)],
                    dst_all)
                fire(0, 0)
                fire(1, 1)

                def body(t, carry):
                    jj = 2 * t
                    for b in range(2):
                        drain(jj + b, b)
                        scat(jj + b, b)
                        fire(jj + b + 2, b)
                    return carry

                lax.fori_loop(0, (nseg_blk - 2) // 2, body, 0)
                for b in range(2):
                    drain(nseg_blk - 2 + b, b)
                    scat(nseg_blk - 2 + b, b)
            plsc.subcore_barrier()
            for c in range(nc):
                @pl.when(cid == c)
                def _():
                    pltpu.sync_copy(
                        shared.at[pl.ds(sid * rows_per_sub, rows_per_sub)],
                        outs[c * nchunks + ch].at[pl.ds(sid * rows_per_sub, rows_per_sub)],
                    )
            plsc.subcore_barrier()

    return segsum_kernel


def _make_batch_gather_kernel(nc, ns, batch, ntab):
    nw = nc * ns
    b_per_w = batch // nw

    out_type = [
        jax.ShapeDtypeStruct((batch, K), jnp.float32) for _ in range(2 * ntab)
    ]

    @functools.partial(
        pl.kernel,
        mesh=_sc_mesh(),
        out_type=out_type,
        compiler_params=pltpu.CompilerParams(use_tc_tiling_on_sc=False),
        scratch_types=[
            pltpu.VMEM((b_per_w,), jnp.int32),
            pltpu.VMEM((b_per_w, K), jnp.float32),
            pltpu.SemaphoreType.DMA,
        ],
    )
    def gather_kernel(*refs):
        tables = refs[:ntab]
        uidx_hbm, iidx_hbm = refs[ntab : ntab + 2]
        outs = refs[ntab + 2 : ntab + 2 + 2 * ntab]
        idx_v, rows_v, sem = refs[ntab + 2 + 2 * ntab :]
        cid = lax.axis_index("c")
        sid = lax.axis_index("s")
        wid = sid * nc + cid
        base = wid * b_per_w
        pltpu.sync_copy(uidx_hbm.at[pl.ds(base, b_per_w)], idx_v)
        for t in range(ntab):
            pltpu.async_copy(tables[t].at[idx_v], rows_v, sem).wait()
            pltpu.sync_copy(rows_v, outs[t].at[pl.ds(base, b_per_w)])
        pltpu.sync_copy(iidx_hbm.at[pl.ds(base, b_per_w)], idx_v)
        for t in range(ntab):
            pltpu.async_copy(tables[t].at[idx_v], rows_v, sem).wait()
            pltpu.sync_copy(rows_v, outs[ntab + t].at[pl.ds(base, b_per_w)])

    return gather_kernel


def _scale_body(x_ref, d0_ref, d1_ref, dinv_ref, *y_refs):
    deg = jnp.maximum(d0_ref[...] + d1_ref[...], 1.0)
    dinv = lax.rsqrt(deg)
    dinv_ref[...] = dinv
    y = x_ref[...] * dinv[:, 0:1]
    for c in range(K // L):
        y_refs[c][...] = y[:, c * L:(c + 1) * L]


def _tc_scale(x0, d0, d1):
    grid = N_NODES // ROWBLK
    rb = lambda i: (i, 0)
    return pl.pallas_call(
        _scale_body,
        grid=(grid,),
        in_specs=[
            pl.BlockSpec((ROWBLK, K), rb),
            pl.BlockSpec((ROWBLK, L), rb),
            pl.BlockSpec((ROWBLK, L), rb),
        ],
        out_specs=[pl.BlockSpec((ROWBLK, L), rb)] * 5,
        out_shape=[jax.ShapeDtypeStruct((N_NODES, L), jnp.float32)] * 5,
    )(x0, d0, d1)


def _make_layer_body(emit_y):
    def body(x_ref, dinv_ref, p00, p01, p02, p03, p10, p11, p12, p13,
             pn0, pn1, w1_ref, w2_ref, bias_ref, *out_refs):
        x = x_ref[...]
        dinvcol = dinv_ref[...][:, 0:1]
        s1 = jnp.concatenate(
            [p00[...] + p10[...], p01[...] + p11[...],
             p02[...] + p12[...], p03[...] + p13[...]], axis=1)
        s1 = s1 * dinvcol
        nsum = (pn0[...][:, 0:1] + pn1[...][:, 0:1]) * dinvcol
        b1 = bias_ref[...][0:1, :]
        bsum = bias_ref[...][1:2, :]
        t = (
            jnp.dot(s1 + x, w1_ref[...], preferred_element_type=jnp.float32)
            + jnp.dot(x * s1, w2_ref[...], preferred_element_type=jnp.float32)
            + nsum * bsum
            + b1
        )
        xn = jnp.where(t >= 0.0, t, 0.2 * t)
        out_refs[0][...] = xn
        if emit_y:
            yn = xn * dinvcol
            for c in range(K // L):
                out_refs[1 + c][...] = yn[:, c * L:(c + 1) * L]
    return body


def _tc_layer(x, dinv16, parts, pn, W1, W2, bias_pack, emit_y):
    grid = N_NODES // ROWBLK
    rb = lambda i: (i, 0)
    zb = lambda i: (0, 0)
    in_specs = (
        [pl.BlockSpec((ROWBLK, K), rb), pl.BlockSpec((ROWBLK, L), rb)]
        + [pl.BlockSpec((ROWBLK, L), rb)] * 10
        + [pl.BlockSpec((K, K), zb)] * 2
        + [pl.BlockSpec((8, K), zb)]
    )
    out_specs = [pl.BlockSpec((ROWBLK, K), rb)]
    out_shape = [jax.ShapeDtypeStruct((N_NODES, K), jnp.float32)]
    if emit_y:
        out_specs += [pl.BlockSpec((ROWBLK, L), rb)] * (K // L)
        out_shape += [jax.ShapeDtypeStruct((N_NODES, L), jnp.float32)] * (K // L)
    res = pl.pallas_call(
        _make_layer_body(emit_y),
        grid=(grid,),
        in_specs=in_specs,
        out_specs=out_specs,
        out_shape=out_shape,
    )(x, dinv16, *parts, *pn, W1, W2, bias_pack)
    return res


def _dot_body(*refs):
    gus = refs[0:3]
    gis = refs[3:6]
    out_ref = refs[6]
    acc = gus[0][...] * gis[0][...]
    acc = acc + gus[1][...] * gis[1][...]
    acc = acc + gus[2][...] * gis[2][...]
    s = jnp.sum(acc, axis=1, keepdims=True)
    out_ref[...] = jnp.broadcast_to(s, out_ref.shape)


def _tc_dot(gus, gis, batch):
    return pl.pallas_call(
        _dot_body,
        grid=(1,),
        in_specs=[pl.BlockSpec((batch, K), lambda i: (0, 0))] * 6,
        out_specs=pl.BlockSpec((batch, 128), lambda i: (0, 0)),
        out_shape=jax.ShapeDtypeStruct((batch, 128), jnp.float32),
    )(*gus, *gis)


def kernel(Gu, Gi, W1_0, b1_0, W2_0, b2_0, W1_1, b1_1, W2_1, b2_1,
           edge_index, user, item):
    nc, ns = _worker_geometry()
    nw = nc * ns
    n_edges = edge_index.shape[1]
    e_sym = 2 * n_edges
    blk_all = nw * EBLK * 2 * NSEG
    e_pad = ((e_sym + blk_all - 1) // blk_all) * blk_all

    u = edge_index[0]
    i_n = edge_index[1] + N_USERS
    src = jnp.concatenate([u, i_n])
    dst = jnp.concatenate([i_n, u])
    pad = e_pad - e_sym
    src_p = jnp.concatenate([src, jnp.zeros((pad,), jnp.int32)]).reshape(-1, EBLK)
    dst_p = jnp.concatenate(
        [dst, jnp.full((pad,), N_NODES, jnp.int32)]).reshape(-1, EBLK)

    zeros_buf = jnp.zeros((N_PAD, L), jnp.float32)
    ones_blk = jnp.ones((EBLK, L), jnp.float32)

    deg_parts = _make_deg_kernel(nc, ns, e_pad)(dst_p, ones_blk, zeros_buf)
    d0 = deg_parts[0]
    d1 = deg_parts[1] if nc > 1 else jnp.zeros_like(d0)
    for c in range(2, nc):
        d1 = d1 + deg_parts[c]

    x0 = jnp.concatenate([Gu, Gi], axis=0)
    dinv16, *y0_chunks = _tc_scale(x0, d0, d1)

    bias_pack0 = jnp.zeros((8, K), jnp.float32).at[0].set(b1_0).at[1].set(b1_0 + b2_0)
    bias_pack1 = jnp.zeros((8, K), jnp.float32).at[0].set(b1_1).at[1].set(b1_1 + b2_1)

    seg5 = _make_segsum_kernel(nc, ns, e_pad, 5)
    parts = seg5(*y0_chunks, dinv16, src_p, dst_p, zeros_buf)
    def pick(parts, nchunks, ch):
        p0 = parts[ch]
        p1 = (parts[nchunks + ch] if nc > 1 else jnp.zeros_like(p0))
        for c in range(2, nc):
            p1 = p1 + parts[c * nchunks + ch]
        return p0, p1

    p_feat0 = [pick(parts, 5, ch)[0] for ch in range(4)]
    p_feat1 = [pick(parts, 5, ch)[1] for ch in range(4)]
    pn0, pn1 = pick(parts, 5, 4)

    x1, *y1_chunks = _tc_layer(
        x0, dinv16, p_feat0 + p_feat1, [pn0, pn1], W1_0, W2_0, bias_pack0, True)

    seg4 = _make_segsum_kernel(nc, ns, e_pad, 4)
    parts2 = seg4(*y1_chunks, src_p, dst_p, zeros_buf)
    q_feat0 = [pick(parts2, 4, ch)[0] for ch in range(4)]
    q_feat1 = [pick(parts2, 4, ch)[1] for ch in range(4)]

    (x2,) = _tc_layer(
        x1, dinv16, q_feat0 + q_feat1, [pn0, pn1], W1_1, W2_1, bias_pack1, False)

    batch = user.shape[0]
    gath = _make_batch_gather_kernel(nc, ns, batch, 3)
    g = gath(x0, x1, x2, user, item + N_USERS)
    gus, gis = g[0:3], g[3:6]
    out = _tc_dot(gus, gis, batch)
    return out[:, 0]


# speedup vs baseline: 1.2640x; 1.0964x over previous
import functools

import jax
import jax.numpy as jnp
from jax import lax
from jax.experimental import pallas as pl
from jax.experimental.pallas import tpu as pltpu
from jax.experimental.pallas import tpu_sc as plsc

N_USERS = 50000
N_ITEMS = 50000
N_NODES = N_USERS + N_ITEMS
K = 64
L = 16
EBLK = 128
NSEG = 4
N_PAD = 100096
ROWBLK = 2000


def _sc_mesh():
    return plsc.VectorSubcoreMesh(core_axis_name="c", subcore_axis_name="s")


def _worker_geometry():
    info = plsc.get_sparse_core_info()
    return info.num_cores, info.num_subcores


def _make_deg_kernel(nc, ns, e_pad):
    e_core = e_pad // nc
    e_sub = e_core // ns
    nblk = e_sub // EBLK
    rows_per_sub = N_PAD // ns

    out_type = [jax.ShapeDtypeStruct((N_PAD, L), jnp.float32) for _ in range(nc)]

    @functools.partial(
        pl.kernel,
        mesh=_sc_mesh(),
        out_type=out_type,
        compiler_params=pltpu.CompilerParams(use_tc_tiling_on_sc=False),
        scratch_types=[
            pltpu.VMEM((e_pad // (nc * ns) // EBLK // NSEG, EBLK), jnp.int32),
            pltpu.VMEM((EBLK, L), jnp.float32),
            pltpu.VMEM_SHARED((N_PAD, L), jnp.float32),
            pltpu.SemaphoreType.DMA,
        ],
    )
    def deg_kernel(dst_hbm, ones_hbm, zeros_hbm, *rest):
        nseg_blk = nblk // NSEG
        outs = rest[:nc]
        dst_all, ones_v, shared, sem = rest[nc:]
        cid = lax.axis_index("c")
        sid = lax.axis_index("s")
        pltpu.sync_copy(
            zeros_hbm.at[pl.ds(sid * rows_per_sub, rows_per_sub)],
            shared.at[pl.ds(sid * rows_per_sub, rows_per_sub)],
        )
        base_blk = (cid * e_core + sid * e_sub) // EBLK
        pltpu.sync_copy(ones_hbm, ones_v)
        plsc.subcore_barrier()

        def body(j, carry):
            pltpu.sync_copy(ones_v, shared.at[dst_all.at[j]], add=True)
            return carry

        for sg in range(NSEG):
            pltpu.sync_copy(
                dst_hbm.at[pl.ds(base_blk + sg * nseg_blk, nseg_blk)], dst_all)
            lax.fori_loop(0, nseg_blk, body, 0)
        plsc.subcore_barrier()
        for c in range(nc):
            @pl.when(cid == c)
            def _():
                pltpu.sync_copy(
                    shared.at[pl.ds(sid * rows_per_sub, rows_per_sub)],
                    outs[c].at[pl.ds(sid * rows_per_sub, rows_per_sub)],
                )

    return deg_kernel


def _make_segsum_kernel(nc, ns, e_pad, nchunks, split_chunks=False):
    if split_chunks:
        e_sub = e_pad // ns
        nseg = NSEG * nc
        n_out = nchunks
    else:
        e_sub = e_pad // nc // ns
        nseg = NSEG
        n_out = nc * nchunks
    nblk = e_sub // EBLK
    rows_per_sub = N_PAD // ns

    out_type = [
        jax.ShapeDtypeStruct((N_PAD, L), jnp.float32) for _ in range(n_out)
    ]

    @functools.partial(
        pl.kernel,
        mesh=_sc_mesh(),
        out_type=out_type,
        compiler_params=pltpu.CompilerParams(use_tc_tiling_on_sc=False),
        scratch_types=[
            pltpu.VMEM((e_pad // (nc * ns) // EBLK // NSEG, EBLK), jnp.int32),
            pltpu.VMEM((e_pad // (nc * ns) // EBLK // NSEG, EBLK), jnp.int32),
            pltpu.VMEM((EBLK, L), jnp.float32),
            pltpu.VMEM((EBLK, L), jnp.float32),
            pltpu.VMEM_SHARED((N_PAD, L), jnp.float32),
            pltpu.SemaphoreType.DMA,
            pltpu.SemaphoreType.DMA,
        ],
    )
    def segsum_kernel(*refs):
        nseg_blk = nblk // nseg
        tables = refs[:nchunks]
        src_hbm, dst_hbm, zeros_hbm = refs[nchunks : nchunks + 3]
        outs = refs[nchunks + 3 : nchunks + 3 + n_out]
        scratch = refs[nchunks + 3 + n_out :]
        src_all, dst_all, rows0, rows1, shared, sem0, sem1 = scratch
        rows = (rows0, rows1)
        sems = (sem0, sem1)
        cid = lax.axis_index("c")
        sid = lax.axis_index("s")
        if split_chunks:
            base_blk = (sid * e_sub) // EBLK
        else:
            base_blk = (cid * (e_pad // nc) + sid * e_sub) // EBLK

        def run_chunk(ch):
            pltpu.sync_copy(
                zeros_hbm.at[pl.ds(sid * rows_per_sub, rows_per_sub)],
                shared.at[pl.ds(sid * rows_per_sub, rows_per_sub)],
            )
            plsc.subcore_barrier()

            def fire(j, b):
                pltpu.async_copy(tables[ch].at[src_all.at[j]], rows[b], sems[b])

            def drain(j, b):
                pltpu.make_async_copy(
                    tables[ch].at[src_all.at[j]], rows[b], sems[b]).wait()

            def scat(j, b):
                pltpu.sync_copy(rows[b], shared.at[dst_all.at[j]], add=True)

            for sg in range(nseg):
                pltpu.sync_copy(
                    src_hbm.at[pl.ds(base_blk + sg * nseg_blk, nseg_blk)],
                    src_all)
                pltpu.sync_copy(
                    dst_hbm.at[pl.ds(base_blk + sg * nseg_blk, nseg_blk)],
                    dst_all)
                fire(0, 0)
                fire(1, 1)

                def body(t, carry):
                    jj = 2 * t
                    for b in range(2):
                        drain(jj + b, b)
                        scat(jj + b, b)
                        fire(jj + b + 2, b)
                    return carry

                lax.fori_loop(0, (nseg_blk - 2) // 2, body, 0)
                for b in range(2):
                    drain(nseg_blk - 2 + b, b)
                    scat(nseg_blk - 2 + b, b)
            plsc.subcore_barrier()
            copyout(ch)
            plsc.subcore_barrier()

        def _store(out_ref):
            pltpu.sync_copy(
                shared.at[pl.ds(sid * rows_per_sub, rows_per_sub)],
                out_ref.at[pl.ds(sid * rows_per_sub, rows_per_sub)],
            )

        if split_chunks:
            per_core = nchunks // nc
            copyout = lambda ch: _store(outs[ch])
            for c in range(nc):
                @pl.when(cid == c)
                def _():
                    for ch in range(c * per_core, (c + 1) * per_core):
                        run_chunk(ch)
        else:
            def copyout(ch):
                for c in range(nc):
                    @pl.when(cid == c)
                    def _():
                        _store(outs[c * nchunks + ch])
            for ch in range(nchunks):
                run_chunk(ch)

    return segsum_kernel


def _make_batch_gather_kernel(nc, ns, batch, ntab):
    nw = nc * ns
    b_per_w = batch // nw

    out_type = [
        jax.ShapeDtypeStruct((batch, K), jnp.float32) for _ in range(2 * ntab)
    ]

    @functools.partial(
        pl.kernel,
        mesh=_sc_mesh(),
        out_type=out_type,
        compiler_params=pltpu.CompilerParams(use_tc_tiling_on_sc=False),
        scratch_types=[
            pltpu.VMEM((b_per_w,), jnp.int32),
            pltpu.VMEM((b_per_w, K), jnp.float32),
            pltpu.SemaphoreType.DMA,
        ],
    )
    def gather_kernel(*refs):
        tables = refs[:ntab]
        uidx_hbm, iidx_hbm = refs[ntab : ntab + 2]
        outs = refs[ntab + 2 : ntab + 2 + 2 * ntab]
        idx_v, rows_v, sem = refs[ntab + 2 + 2 * ntab :]
        cid = lax.axis_index("c")
        sid = lax.axis_index("s")
        wid = sid * nc + cid
        base = wid * b_per_w
        pltpu.sync_copy(uidx_hbm.at[pl.ds(base, b_per_w)], idx_v)
        for t in range(ntab):
            pltpu.async_copy(tables[t].at[idx_v], rows_v, sem).wait()
            pltpu.sync_copy(rows_v, outs[t].at[pl.ds(base, b_per_w)])
        pltpu.sync_copy(iidx_hbm.at[pl.ds(base, b_per_w)], idx_v)
        for t in range(ntab):
            pltpu.async_copy(tables[t].at[idx_v], rows_v, sem).wait()
            pltpu.sync_copy(rows_v, outs[ntab + t].at[pl.ds(base, b_per_w)])

    return gather_kernel


def _scale_body(x_ref, d0_ref, d1_ref, dinv_ref, *y_refs):
    deg = jnp.maximum(d0_ref[...] + d1_ref[...], 1.0)
    dinv = lax.rsqrt(deg)
    dinv_ref[...] = dinv
    y = x_ref[...] * dinv[:, 0:1]
    for c in range(K // L):
        y_refs[c][...] = y[:, c * L:(c + 1) * L]


def _tc_scale(x0, d0, d1):
    grid = N_NODES // ROWBLK
    rb = lambda i: (i, 0)
    return pl.pallas_call(
        _scale_body,
        grid=(grid,),
        in_specs=[
            pl.BlockSpec((ROWBLK, K), rb),
            pl.BlockSpec((ROWBLK, L), rb),
            pl.BlockSpec((ROWBLK, L), rb),
        ],
        out_specs=[pl.BlockSpec((ROWBLK, L), rb)] * 5,
        out_shape=[jax.ShapeDtypeStruct((N_NODES, L), jnp.float32)] * 5,
    )(x0, d0, d1)


def _make_layer_body(emit_y, n_feat):
    def body(x_ref, dinv_ref, *rest):
        parts = rest[:n_feat]
        pn0, pn1, w1_ref, w2_ref, bias_ref = rest[n_feat : n_feat + 5]
        out_refs = rest[n_feat + 5 :]
        x = x_ref[...]
        dinvcol = dinv_ref[...][:, 0:1]
        if n_feat == 8:
            cols = [parts[c][...] + parts[4 + c][...] for c in range(4)]
        else:
            cols = [parts[c][...] for c in range(4)]
        s1 = jnp.concatenate(cols, axis=1) * dinvcol
        nsum = (pn0[...][:, 0:1] + pn1[...][:, 0:1]) * dinvcol
        b1 = bias_ref[...][0:1, :]
        bsum = bias_ref[...][1:2, :]
        t = (
            jnp.dot(s1 + x, w1_ref[...], preferred_element_type=jnp.float32)
            + jnp.dot(x * s1, w2_ref[...], preferred_element_type=jnp.float32)
            + nsum * bsum
            + b1
        )
        xn = jnp.where(t >= 0.0, t, 0.2 * t)
        out_refs[0][...] = xn
        if emit_y:
            yn = xn * dinvcol
            for c in range(K // L):
                out_refs[1 + c][...] = yn[:, c * L:(c + 1) * L]
    return body


def _tc_layer(x, dinv16, parts, pn, W1, W2, bias_pack, emit_y):
    grid = N_NODES // ROWBLK
    rb = lambda i: (i, 0)
    zb = lambda i: (0, 0)
    in_specs = (
        [pl.BlockSpec((ROWBLK, K), rb), pl.BlockSpec((ROWBLK, L), rb)]
        + [pl.BlockSpec((ROWBLK, L), rb)] * (len(parts) + 2)
        + [pl.BlockSpec((K, K), zb)] * 2
        + [pl.BlockSpec((8, K), zb)]
    )
    out_specs = [pl.BlockSpec((ROWBLK, K), rb)]
    out_shape = [jax.ShapeDtypeStruct((N_NODES, K), jnp.float32)]
    if emit_y:
        out_specs += [pl.BlockSpec((ROWBLK, L), rb)] * (K // L)
        out_shape += [jax.ShapeDtypeStruct((N_NODES, L), jnp.float32)] * (K // L)
    res = pl.pallas_call(
        _make_layer_body(emit_y, len(parts)),
        grid=(grid,),
        in_specs=in_specs,
        out_specs=out_specs,
        out_shape=out_shape,
    )(x, dinv16, *parts, *pn, W1, W2, bias_pack)
    return res


def _dot_body(*refs):
    gus = refs[0:3]
    gis = refs[3:6]
    out_ref = refs[6]
    acc = gus[0][...] * gis[0][...]
    acc = acc + gus[1][...] * gis[1][...]
    acc = acc + gus[2][...] * gis[2][...]
    s = jnp.sum(acc, axis=1, keepdims=True)
    out_ref[...] = jnp.broadcast_to(s, out_ref.shape)


def _tc_dot(gus, gis, batch):
    return pl.pallas_call(
        _dot_body,
        grid=(1,),
        in_specs=[pl.BlockSpec((batch, K), lambda i: (0, 0))] * 6,
        out_specs=pl.BlockSpec((batch, 128), lambda i: (0, 0)),
        out_shape=jax.ShapeDtypeStruct((batch, 128), jnp.float32),
    )(*gus, *gis)


def kernel(Gu, Gi, W1_0, b1_0, W2_0, b2_0, W1_1, b1_1, W2_1, b2_1,
           edge_index, user, item):
    nc, ns = _worker_geometry()
    nw = nc * ns
    n_edges = edge_index.shape[1]
    e_sym = 2 * n_edges
    blk_all = nw * EBLK * 2 * NSEG
    e_pad = ((e_sym + blk_all - 1) // blk_all) * blk_all

    u = edge_index[0]
    i_n = edge_index[1] + N_USERS
    src = jnp.concatenate([u, i_n])
    dst = jnp.concatenate([i_n, u])
    pad = e_pad - e_sym
    src_p = jnp.concatenate([src, jnp.zeros((pad,), jnp.int32)]).reshape(-1, EBLK)
    dst_p = jnp.concatenate(
        [dst, jnp.full((pad,), N_NODES, jnp.int32)]).reshape(-1, EBLK)

    zeros_buf = jnp.zeros((N_PAD, L), jnp.float32)
    ones_blk = jnp.ones((EBLK, L), jnp.float32)

    deg_parts = _make_deg_kernel(nc, ns, e_pad)(dst_p, ones_blk, zeros_buf)
    d0 = deg_parts[0]
    d1 = deg_parts[1] if nc > 1 else jnp.zeros_like(d0)
    for c in range(2, nc):
        d1 = d1 + deg_parts[c]

    x0 = jnp.concatenate([Gu, Gi], axis=0)
    dinv16, *y0_chunks = _tc_scale(x0, d0, d1)

    bias_pack0 = jnp.zeros((8, K), jnp.float32).at[0].set(b1_0).at[1].set(b1_0 + b2_0)
    bias_pack1 = jnp.zeros((8, K), jnp.float32).at[0].set(b1_1).at[1].set(b1_1 + b2_1)

    seg5 = _make_segsum_kernel(nc, ns, e_pad, 5)
    parts = seg5(*y0_chunks, dinv16, src_p, dst_p, zeros_buf)
    def pick(parts, nchunks, ch):
        p0 = parts[ch]
        p1 = (parts[nchunks + ch] if nc > 1 else jnp.zeros_like(p0))
        for c in range(2, nc):
            p1 = p1 + parts[c * nchunks + ch]
        return p0, p1

    p_feat0 = [pick(parts, 5, ch)[0] for ch in range(4)]
    p_feat1 = [pick(parts, 5, ch)[1] for ch in range(4)]
    pn0, pn1 = pick(parts, 5, 4)

    x1, *y1_chunks = _tc_layer(
        x0, dinv16, p_feat0 + p_feat1, [pn0, pn1], W1_0, W2_0, bias_pack0, True)

    seg4 = _make_segsum_kernel(nc, ns, e_pad, 4, split_chunks=(nc == 2))
    parts2 = seg4(*y1_chunks, src_p, dst_p, zeros_buf)
    if nc == 2:
        q_feats = list(parts2)
    else:
        q_feats = ([pick(parts2, 4, ch)[0] for ch in range(4)]
                   + [pick(parts2, 4, ch)[1] for ch in range(4)])

    (x2,) = _tc_layer(
        x1, dinv16, q_feats, [pn0, pn1], W1_1, W2_1, bias_pack1, False)

    batch = user.shape[0]
    gath = _make_batch_gather_kernel(nc, ns, batch, 3)
    g = gath(x0, x1, x2, user, item + N_USERS)
    gus, gis = g[0:3], g[3:6]
    out = _tc_dot(gus, gis, batch)
    return out[:, 0]


# speedup vs baseline: 1.3368x; 1.0576x over previous
import functools

import jax
import jax.numpy as jnp
from jax import lax
from jax.experimental import pallas as pl
from jax.experimental.pallas import tpu as pltpu
from jax.experimental.pallas import tpu_sc as plsc

N_USERS = 50000
N_ITEMS = 50000
N_NODES = N_USERS + N_ITEMS
K = 64
L = 16
EBLK = 128
NSEG = 4
N_PAD = 100096
ROWBLK = 2000


def _sc_mesh():
    return plsc.VectorSubcoreMesh(core_axis_name="c", subcore_axis_name="s")


def _worker_geometry():
    info = plsc.get_sparse_core_info()
    return info.num_cores, info.num_subcores


def _make_deg_kernel(nc, ns, e_pad):
    e_core = e_pad // nc
    e_sub = e_core // ns
    nblk = e_sub // EBLK
    rows_per_sub = N_PAD // ns

    out_type = [jax.ShapeDtypeStruct((N_PAD, L), jnp.float32) for _ in range(nc)]

    @functools.partial(
        pl.kernel,
        mesh=_sc_mesh(),
        out_type=out_type,
        compiler_params=pltpu.CompilerParams(use_tc_tiling_on_sc=False),
        scratch_types=[
            pltpu.VMEM((e_pad // (nc * ns) // EBLK // NSEG, EBLK), jnp.int32),
            pltpu.VMEM((EBLK, L), jnp.float32),
            pltpu.VMEM_SHARED((N_PAD, L), jnp.float32),
            pltpu.SemaphoreType.DMA,
        ],
    )
    def deg_kernel(dst_hbm, ones_hbm, zeros_hbm, *rest):
        nseg_blk = nblk // NSEG
        outs = rest[:nc]
        dst_all, ones_v, shared, sem = rest[nc:]
        cid = lax.axis_index("c")
        sid = lax.axis_index("s")
        pltpu.sync_copy(
            zeros_hbm.at[pl.ds(sid * rows_per_sub, rows_per_sub)],
            shared.at[pl.ds(sid * rows_per_sub, rows_per_sub)],
        )
        base_blk = (cid * e_core + sid * e_sub) // EBLK
        pltpu.sync_copy(ones_hbm, ones_v)
        plsc.subcore_barrier()

        def body(j, carry):
            pltpu.sync_copy(ones_v, shared.at[dst_all.at[j]], add=True)
            return carry

        for sg in range(NSEG):
            pltpu.sync_copy(
                dst_hbm.at[pl.ds(base_blk + sg * nseg_blk, nseg_blk)], dst_all)
            lax.fori_loop(0, nseg_blk, body, 0)
        plsc.subcore_barrier()
        for c in range(nc):
            @pl.when(cid == c)
            def _():
                pltpu.sync_copy(
                    shared.at[pl.ds(sid * rows_per_sub, rows_per_sub)],
                    outs[c].at[pl.ds(sid * rows_per_sub, rows_per_sub)],
                )

    return deg_kernel


def _make_segsum_kernel(nc, ns, e_pad, nchunks, split_chunks=False):
    if split_chunks:
        e_sub = e_pad // ns
        nseg = NSEG * nc
        n_out = nchunks
    else:
        e_sub = e_pad // nc // ns
        nseg = NSEG
        n_out = nc * nchunks
    nblk = e_sub // EBLK
    rows_per_sub = N_PAD // ns

    out_type = [
        jax.ShapeDtypeStruct((N_PAD, L), jnp.float32) for _ in range(n_out)
    ]

    @functools.partial(
        pl.kernel,
        mesh=_sc_mesh(),
        out_type=out_type,
        compiler_params=pltpu.CompilerParams(use_tc_tiling_on_sc=False),
        scratch_types=[
            pltpu.VMEM((e_pad // (nc * ns) // EBLK // NSEG, EBLK), jnp.int32),
            pltpu.VMEM((e_pad // (nc * ns) // EBLK // NSEG, EBLK), jnp.int32),
            pltpu.VMEM((EBLK, L), jnp.float32),
            pltpu.VMEM((EBLK, L), jnp.float32),
            pltpu.VMEM_SHARED((N_PAD, L), jnp.float32),
            pltpu.SemaphoreType.DMA,
            pltpu.SemaphoreType.DMA,
        ],
    )
    def segsum_kernel(*refs):
        nseg_blk = nblk // nseg
        tables = refs[:nchunks]
        src_hbm, dst_hbm, zeros_hbm = refs[nchunks : nchunks + 3]
        outs = refs[nchunks + 3 : nchunks + 3 + n_out]
        scratch = refs[nchunks + 3 + n_out :]
        src_all, dst_all, rows0, rows1, shared, sem0, sem1 = scratch
        rows = (rows0, rows1)
        sems = (sem0, sem1)
        cid = lax.axis_index("c")
        sid = lax.axis_index("s")
        if split_chunks:
            base_blk = (sid * e_sub) // EBLK
        else:
            base_blk = (cid * (e_pad // nc) + sid * e_sub) // EBLK

        def run_chunk(ch):
            pltpu.sync_copy(
                zeros_hbm.at[pl.ds(sid * rows_per_sub, rows_per_sub)],
                shared.at[pl.ds(sid * rows_per_sub, rows_per_sub)],
            )
            plsc.subcore_barrier()

            def fire(j, b):
                pltpu.async_copy(tables[ch].at[src_all.at[j]], rows[b], sems[b])

            def drain(j, b):
                pltpu.make_async_copy(
                    tables[ch].at[src_all.at[j]], rows[b], sems[b]).wait()

            def scat(j, b):
                pltpu.sync_copy(rows[b], shared.at[dst_all.at[j]], add=True)

            for sg in range(nseg):
                pltpu.sync_copy(
                    src_hbm.at[pl.ds(base_blk + sg * nseg_blk, nseg_blk)],
                    src_all)
                pltpu.sync_copy(
                    dst_hbm.at[pl.ds(base_blk + sg * nseg_blk, nseg_blk)],
                    dst_all)
                fire(0, 0)
                fire(1, 1)

                def body(t, carry):
                    jj = 2 * t
                    for b in range(2):
                        drain(jj + b, b)
                        scat(jj + b, b)
                        fire(jj + b + 2, b)
                    return carry

                lax.fori_loop(0, (nseg_blk - 2) // 2, body, 0)
                for b in range(2):
                    drain(nseg_blk - 2 + b, b)
                    scat(nseg_blk - 2 + b, b)
            plsc.subcore_barrier()
            copyout(ch)
            plsc.subcore_barrier()

        def _store(out_ref):
            pltpu.sync_copy(
                shared.at[pl.ds(sid * rows_per_sub, rows_per_sub)],
                out_ref.at[pl.ds(sid * rows_per_sub, rows_per_sub)],
            )

        if split_chunks:
            bounds = [i * nchunks // nc for i in range(nc + 1)]
            copyout = lambda ch: _store(outs[ch])
            for c in range(nc):
                @pl.when(cid == c)
                def _():
                    for ch in range(bounds[c], bounds[c + 1]):
                        run_chunk(ch)
        else:
            def copyout(ch):
                for c in range(nc):
                    @pl.when(cid == c)
                    def _():
                        _store(outs[c * nchunks + ch])
            for ch in range(nchunks):
                run_chunk(ch)

    return segsum_kernel


def _make_batch_gather_kernel(nc, ns, batch, ntab):
    nw = nc * ns
    b_per_w = batch // nw

    out_type = [
        jax.ShapeDtypeStruct((batch, K), jnp.float32) for _ in range(2 * ntab)
    ]

    @functools.partial(
        pl.kernel,
        mesh=_sc_mesh(),
        out_type=out_type,
        compiler_params=pltpu.CompilerParams(use_tc_tiling_on_sc=False),
        scratch_types=[
            pltpu.VMEM((b_per_w,), jnp.int32),
            pltpu.VMEM((b_per_w, K), jnp.float32),
            pltpu.SemaphoreType.DMA,
        ],
    )
    def gather_kernel(*refs):
        tables = refs[:ntab]
        uidx_hbm, iidx_hbm = refs[ntab : ntab + 2]
        outs = refs[ntab + 2 : ntab + 2 + 2 * ntab]
        idx_v, rows_v, sem = refs[ntab + 2 + 2 * ntab :]
        cid = lax.axis_index("c")
        sid = lax.axis_index("s")
        wid = sid * nc + cid
        base = wid * b_per_w
        pltpu.sync_copy(uidx_hbm.at[pl.ds(base, b_per_w)], idx_v)
        for t in range(ntab):
            pltpu.async_copy(tables[t].at[idx_v], rows_v, sem).wait()
            pltpu.sync_copy(rows_v, outs[t].at[pl.ds(base, b_per_w)])
        pltpu.sync_copy(iidx_hbm.at[pl.ds(base, b_per_w)], idx_v)
        for t in range(ntab):
            pltpu.async_copy(tables[t].at[idx_v], rows_v, sem).wait()
            pltpu.sync_copy(rows_v, outs[ntab + t].at[pl.ds(base, b_per_w)])

    return gather_kernel


def _scale_body(x_ref, d0_ref, d1_ref, dinv_ref, *y_refs):
    deg = jnp.maximum(d0_ref[...] + d1_ref[...], 1.0)
    dinv = lax.rsqrt(deg)
    dinv_ref[...] = dinv
    y = x_ref[...] * dinv[:, 0:1]
    for c in range(K // L):
        y_refs[c][...] = y[:, c * L:(c + 1) * L]


def _tc_scale(x0, d0, d1):
    grid = N_NODES // ROWBLK
    rb = lambda i: (i, 0)
    return pl.pallas_call(
        _scale_body,
        grid=(grid,),
        in_specs=[
            pl.BlockSpec((ROWBLK, K), rb),
            pl.BlockSpec((ROWBLK, L), rb),
            pl.BlockSpec((ROWBLK, L), rb),
        ],
        out_specs=[pl.BlockSpec((ROWBLK, L), rb)] * 5,
        out_shape=[jax.ShapeDtypeStruct((N_NODES, L), jnp.float32)] * 5,
    )(x0, d0, d1)


def _make_layer_body(emit_y, n_feat):
    def body(x_ref, dinv_ref, *rest):
        parts = rest[:n_feat]
        pn0, pn1, w1_ref, w2_ref, bias_ref = rest[n_feat : n_feat + 5]
        out_refs = rest[n_feat + 5 :]
        x = x_ref[...]
        dinvcol = dinv_ref[...][:, 0:1]
        if n_feat == 8:
            cols = [parts[c][...] + parts[4 + c][...] for c in range(4)]
        else:
            cols = [parts[c][...] for c in range(4)]
        s1 = jnp.concatenate(cols, axis=1) * dinvcol
        nsum = (pn0[...][:, 0:1] + pn1[...][:, 0:1]) * dinvcol
        b1 = bias_ref[...][0:1, :]
        bsum = bias_ref[...][1:2, :]
        t = (
            jnp.dot(s1 + x, w1_ref[...], preferred_element_type=jnp.float32)
            + jnp.dot(x * s1, w2_ref[...], preferred_element_type=jnp.float32)
            + nsum * bsum
            + b1
        )
        xn = jnp.where(t >= 0.0, t, 0.2 * t)
        out_refs[0][...] = xn
        if emit_y:
            yn = xn * dinvcol
            for c in range(K // L):
                out_refs[1 + c][...] = yn[:, c * L:(c + 1) * L]
    return body


def _tc_layer(x, dinv16, parts, pn, W1, W2, bias_pack, emit_y):
    grid = N_NODES // ROWBLK
    rb = lambda i: (i, 0)
    zb = lambda i: (0, 0)
    in_specs = (
        [pl.BlockSpec((ROWBLK, K), rb), pl.BlockSpec((ROWBLK, L), rb)]
        + [pl.BlockSpec((ROWBLK, L), rb)] * (len(parts) + 2)
        + [pl.BlockSpec((K, K), zb)] * 2
        + [pl.BlockSpec((8, K), zb)]
    )
    out_specs = [pl.BlockSpec((ROWBLK, K), rb)]
    out_shape = [jax.ShapeDtypeStruct((N_NODES, K), jnp.float32)]
    if emit_y:
        out_specs += [pl.BlockSpec((ROWBLK, L), rb)] * (K // L)
        out_shape += [jax.ShapeDtypeStruct((N_NODES, L), jnp.float32)] * (K // L)
    res = pl.pallas_call(
        _make_layer_body(emit_y, len(parts)),
        grid=(grid,),
        in_specs=in_specs,
        out_specs=out_specs,
        out_shape=out_shape,
    )(x, dinv16, *parts, *pn, W1, W2, bias_pack)
    return res


def _dot_body(*refs):
    gus = refs[0:3]
    gis = refs[3:6]
    out_ref = refs[6]
    acc = gus[0][...] * gis[0][...]
    acc = acc + gus[1][...] * gis[1][...]
    acc = acc + gus[2][...] * gis[2][...]
    s = jnp.sum(acc, axis=1, keepdims=True)
    out_ref[...] = jnp.broadcast_to(s, out_ref.shape)


def _tc_dot(gus, gis, batch):
    return pl.pallas_call(
        _dot_body,
        grid=(1,),
        in_specs=[pl.BlockSpec((batch, K), lambda i: (0, 0))] * 6,
        out_specs=pl.BlockSpec((batch, 128), lambda i: (0, 0)),
        out_shape=jax.ShapeDtypeStruct((batch, 128), jnp.float32),
    )(*gus, *gis)


def kernel(Gu, Gi, W1_0, b1_0, W2_0, b2_0, W1_1, b1_1, W2_1, b2_1,
           edge_index, user, item):
    nc, ns = _worker_geometry()
    nw = nc * ns
    n_edges = edge_index.shape[1]
    e_sym = 2 * n_edges
    blk_all = nw * EBLK * 2 * NSEG
    e_pad = ((e_sym + blk_all - 1) // blk_all) * blk_all

    u = edge_index[0]
    i_n = edge_index[1] + N_USERS
    src = jnp.concatenate([u, i_n])
    dst = jnp.concatenate([i_n, u])
    pad = e_pad - e_sym
    src_p = jnp.concatenate([src, jnp.zeros((pad,), jnp.int32)]).reshape(-1, EBLK)
    dst_p = jnp.concatenate(
        [dst, jnp.full((pad,), N_NODES, jnp.int32)]).reshape(-1, EBLK)

    zeros_buf = jnp.zeros((N_PAD, L), jnp.float32)
    ones_blk = jnp.ones((EBLK, L), jnp.float32)

    deg_parts = _make_deg_kernel(nc, ns, e_pad)(dst_p, ones_blk, zeros_buf)
    d0 = deg_parts[0]
    d1 = deg_parts[1] if nc > 1 else jnp.zeros_like(d0)
    for c in range(2, nc):
        d1 = d1 + deg_parts[c]

    x0 = jnp.concatenate([Gu, Gi], axis=0)
    dinv16, *y0_chunks = _tc_scale(x0, d0, d1)

    bias_pack0 = jnp.zeros((8, K), jnp.float32).at[0].set(b1_0).at[1].set(b1_0 + b2_0)
    bias_pack1 = jnp.zeros((8, K), jnp.float32).at[0].set(b1_1).at[1].set(b1_1 + b2_1)

    def pick(parts, nchunks, ch):
        p0 = parts[ch]
        p1 = (parts[nchunks + ch] if nc > 1 else jnp.zeros_like(p0))
        for c in range(2, nc):
            p1 = p1 + parts[c * nchunks + ch]
        return p0, p1

    seg5 = _make_segsum_kernel(nc, ns, e_pad, 5, split_chunks=(nc == 2))
    parts = seg5(*y0_chunks, dinv16, src_p, dst_p, zeros_buf)
    if nc == 2:
        p_feats = list(parts[0:4])
        pn0, pn1 = parts[4], zeros_buf
    else:
        p_feats = ([pick(parts, 5, ch)[0] for ch in range(4)]
                   + [pick(parts, 5, ch)[1] for ch in range(4)])
        pn0, pn1 = pick(parts, 5, 4)

    x1, *y1_chunks = _tc_layer(
        x0, dinv16, p_feats, [pn0, pn1], W1_0, W2_0, bias_pack0, True)

    seg4 = _make_segsum_kernel(nc, ns, e_pad, 4, split_chunks=(nc == 2))
    parts2 = seg4(*y1_chunks, src_p, dst_p, zeros_buf)
    if nc == 2:
        q_feats = list(parts2)
    else:
        q_feats = ([pick(parts2, 4, ch)[0] for ch in range(4)]
                   + [pick(parts2, 4, ch)[1] for ch in range(4)])

    (x2,) = _tc_layer(
        x1, dinv16, q_feats, [pn0, pn1], W1_1, W2_1, bias_pack1, False)

    batch = user.shape[0]
    gath = _make_batch_gather_kernel(nc, ns, batch, 3)
    g = gath(x0, x1, x2, user, item + N_USERS)
    gus, gis = g[0:3], g[3:6]
    out = _tc_dot(gus, gis, batch)
    return out[:, 0]
